# Initial kernel scaffold; baseline (speedup 1.0000x reference)
import jax
import jax.numpy as jnp
from jax.experimental import pallas as pl


def kernel(t, state_H, state_K, ind_K, ind_HK, weights, omega, w1, b1, w2):
    raise NotImplementedError("write your pallas kernel here")


# speedup vs baseline: 1.0138x; 1.0138x over previous
import functools

import jax
import jax.numpy as jnp
from jax.experimental import pallas as pl
from jax.experimental.pallas import tpu as pltpu

_N = 10000
_D = 128
_H_MLP = 64
_EPS_K = 0.1
_KAPPA_K = 1.0
_KAPPA_H = 1.0

_BM = 200


def _symmv_body(w_ref, grow_ref, gcol_ref, y1_ref, y2_ref):
    i = pl.program_id(0)

    @pl.when(i == 0)
    def _():
        y2_ref[...] = jnp.zeros_like(y2_ref)

    wblk = w_ref[...]
    y1 = jnp.dot(wblk, gcol_ref[...], preferred_element_type=jnp.float32)
    y1_ref[...] = y1.reshape(1, 1, _BM)
    grow = grow_ref[0, 0, :]
    y2_ref[...] += jnp.dot(grow, wblk, preferred_element_type=jnp.float32)[None, :]


def _sym_matvec(weights, g):
    ni = _N // _BM
    g3 = g.reshape(ni, 1, _BM)
    y1, y2 = pl.pallas_call(
        _symmv_body,
        grid=(ni,),
        in_specs=[
            pl.BlockSpec((_BM, _N), lambda i: (i, 0)),
            pl.BlockSpec((1, 1, _BM), lambda i: (i, 0, 0)),
            pl.BlockSpec((_N,), lambda i: (0,)),
        ],
        out_specs=[
            pl.BlockSpec((1, 1, _BM), lambda i: (i, 0, 0)),
            pl.BlockSpec((1, _N), lambda i: (0, 0)),
        ],
        out_shape=[
            jax.ShapeDtypeStruct((ni, 1, _BM), jnp.float32),
            jax.ShapeDtypeStruct((1, _N), jnp.float32),
        ],
    )(weights, g3, g)
    return 0.5 * (y1.reshape(_N) + y2.reshape(_N))


def kernel(t, state_H, state_K, ind_K, ind_HK, weights, omega, w1, b1, w2):
    sK = state_K / jnp.linalg.norm(state_K, axis=1, keepdims=True)
    g = jnp.tanh(state_H)
    Wg = _sym_matvec(weights, g)

    i0 = ind_HK[:, 0]
    i1 = ind_HK[:, 1]
    j0 = ind_K[:, 0]
    j1 = ind_K[:, 1]

    wf = weights.reshape(-1)
    we = 0.5 * (wf[i0 * _N + i1] + wf[i1 * _N + i0])

    sk_i0 = sK[i0]
    sk_i1 = sK[i1]
    Gram = jnp.sum(sk_i0 * sk_i1, axis=1)
    g0 = g[i0]
    g1 = g[i1]

    f_H = -state_H + Wg
    f_H = f_H.at[i0].add(Gram * we * g1 / _KAPPA_H)
    f_H = f_H.at[i1].add(Gram * we * g0 / _KAPPA_H)

    sk_j0 = sK[j0]
    sk_j1 = sK[j1]
    s = jnp.sum(sk_j0 * sk_j1, axis=1)
    h = jnp.tanh(s[:, None] * w1 + b1)
    c = _EPS_K * (h @ w2)
    f_K = jnp.zeros_like(sK)
    f_K = f_K.at[j0].add(c * sk_j1)
    f_K = f_K.at[j1].add(c * sk_j0)

    coef = (-(g0 * g1) * we / _KAPPA_K)[:, None]
    f_K = f_K.at[i0].add(coef * sk_i1)
    f_K = f_K.at[i1].add(coef * sk_i0)

    f_K = (
        -f_K
        + sK * jnp.sum(sK * f_K, axis=1, keepdims=True)
        + sK @ ((omega - omega.T) / 2.0)
    )
    return (f_H, f_K)


# speedup vs baseline: 2.6938x; 2.6571x over previous
import functools

import jax
import jax.numpy as jnp
from jax import lax
from jax.experimental import pallas as pl
from jax.experimental.pallas import tpu as pltpu
from jax.experimental.pallas import tpu_sc as plsc

_N = 10000
_D = 128
_H_MLP = 64
_EPS_K = 0.1
_KAPPA_K = 1.0
_KAPPA_H = 1.0

_BM = 200


def _symmv_body(w_ref, grow_ref, gcol_ref, y1_ref, y2_ref):
    i = pl.program_id(0)

    @pl.when(i == 0)
    def _():
        y2_ref[...] = jnp.zeros_like(y2_ref)

    wblk = w_ref[...]
    y1 = jnp.dot(wblk, gcol_ref[...], preferred_element_type=jnp.float32)
    y1_ref[...] = y1.reshape(1, 1, _BM)
    grow = grow_ref[0, 0, :]
    y2_ref[...] += jnp.dot(grow, wblk, preferred_element_type=jnp.float32)[None, :]


def _sym_matvec(weights, g):
    ni = _N // _BM
    g3 = g.reshape(ni, 1, _BM)
    y1, y2 = pl.pallas_call(
        _symmv_body,
        grid=(ni,),
        in_specs=[
            pl.BlockSpec((_BM, _N), lambda i: (i, 0)),
            pl.BlockSpec((1, 1, _BM), lambda i: (i, 0, 0)),
            pl.BlockSpec((_N,), lambda i: (0,)),
        ],
        out_specs=[
            pl.BlockSpec((1, 1, _BM), lambda i: (i, 0, 0)),
            pl.BlockSpec((1, _N), lambda i: (0, 0)),
        ],
        out_shape=[
            jax.ShapeDtypeStruct((ni, 1, _BM), jnp.float32),
            jax.ShapeDtypeStruct((1, _N), jnp.float32),
        ],
    )(weights, g3, g)
    return 0.5 * (y1.reshape(_N) + y2.reshape(_N))


_NP = 10240
_EPT = 10000
_C = 80
_CP = 128
_NCH = _EPT // _C
_RPT = _NP // 16


_DUMP = 10100


def _edge_body(sk_hbm, g_hbm, wflat_hbm, i0_hbm, i1_hbm, j0_hbm, j1_hbm,
               w1d_hbm, b2_hbm, w2_hbm, zfh_hbm, zfk_hbm,
               fh_out, fk_out,
               g_tab, fh_acc, idx0, idx1, idxp0, idxp1,
               fl01, fl10, w01, w10,
               rows0, rows1, red_v, w1v, b2v, w2v, fk_sh, sem):
    cid = lax.axis_index("c")
    sid = lax.axis_index("s")
    wid = sid * 2 + cid

    pltpu.sync_copy(g_hbm, g_tab)
    pltpu.sync_copy(w1d_hbm, w1v)
    pltpu.sync_copy(b2_hbm, b2v)
    pltpu.sync_copy(w2_hbm, w2v)
    pltpu.sync_copy(zfh_hbm, fh_acc)
    pltpu.sync_copy(zfk_hbm, fk_sh.at[pl.ds(sid * _RPT, _RPT)])
    plsc.subcore_barrier()

    def _hsum(v):
        for sh in (8, 4, 2, 1):
            red_v[pl.ds(0, 16)] = v
            v = v + red_v[pl.ds(sh, 16)]
        return v[0]

    lane0 = lax.iota(jnp.int32, 16) == 0
    w1g = [w1v[pl.ds(16 * i, 16)] for i in range(_H_MLP // 16)]
    b2g = [b2v[pl.ds(16 * i, 16)] for i in range(_H_MLP // 16)]
    w2g = [w2v[pl.ds(16 * i, 16)] for i in range(_H_MLP // 16)]

    def gather_chunk(k, ia_hbm, ib_hbm, with_w):
        base = wid * _EPT + k * _C
        pltpu.sync_copy(ia_hbm.at[pl.ds(base, _C)], idx0)
        pltpu.sync_copy(ib_hbm.at[pl.ds(base, _C)], idx1)
        pltpu.sync_copy(ia_hbm.at[pl.ds(base, _C + 16)], idxp0)
        pltpu.sync_copy(ib_hbm.at[pl.ds(base, _C + 16)], idxp1)
        cps = [pltpu.async_copy(sk_hbm.at[idx0], rows0, sem),
               pltpu.async_copy(sk_hbm.at[idx1], rows1, sem)]
        if with_w:
            def mkflat(gi, _):
                a = idxp0[pl.ds(gi * 16, 16)]
                b = idxp1[pl.ds(gi * 16, 16)]
                fl01[pl.ds(gi * 16, 16)] = a * _N + b
                fl10[pl.ds(gi * 16, 16)] = b * _N + a
                return 0
            lax.fori_loop(0, (_C + 16) // 16, mkflat, 0)
            cps.append(pltpu.async_copy(wflat_hbm.at[fl01], w01, sem))
            cps.append(pltpu.async_copy(wflat_hbm.at[fl10], w10, sem))
        for cp in cps:
            cp.wait()

    def edge_dot(e):
        acc = rows0[e, pl.ds(0, 16)] * rows1[e, pl.ds(0, 16)]
        for kk in range(1, _D // 16):
            sl = pl.ds(kk * 16, 16)
            acc = acc + rows0[e, sl] * rows1[e, sl]
        return _hsum(acc)

    def scale_rows(e, ck):
        ckv = jnp.full((16,), ck, jnp.float32)
        for kk in range(_D // 16):
            sl = pl.ds(kk * 16, 16)
            rows1[e, sl] = rows1[e, sl] * ckv
            rows0[e, sl] = rows0[e, sl] * ckv

    def scatter_fk():
        pltpu.sync_copy(rows1, fk_sh.at[idx0], add=True)
        pltpu.sync_copy(rows0, fk_sh.at[idx1], add=True)

    def hk_chunk(k, _):
        gather_chunk(k, i0_hbm, i1_hbm, True)

        def edge(e, _):
            gram = edge_dot(e)
            i0s = idxp0[pl.ds(e, 16)][0]
            i1s = idxp1[pl.ds(e, 16)][0]
            g0 = g_tab[pl.ds(i0s, 16)][0]
            g1 = g_tab[pl.ds(i1s, 16)][0]
            we = 0.5 * (w01[pl.ds(e, 16)][0] + w10[pl.ds(e, 16)][0])
            gw = gram * we * (1.0 / _KAPPA_H)
            v0 = fh_acc[pl.ds(i0s, 16)]
            fh_acc[pl.ds(i0s, 16)] = v0 + jnp.where(lane0, gw * g1, 0.0)
            v1 = fh_acc[pl.ds(i1s, 16)]
            fh_acc[pl.ds(i1s, 16)] = v1 + jnp.where(lane0, gw * g0, 0.0)
            scale_rows(e, -(g0 * g1) * we * (1.0 / _KAPPA_K))
            return 0

        lax.fori_loop(0, _C, edge, 0)
        scatter_fk()
        return 0

    def k_chunk(k, _):
        gather_chunk(k, j0_hbm, j1_hbm, False)

        def edge(e, _):
            s2 = 2.0 * edge_dot(e)
            macc = jnp.zeros((16,), jnp.float32)
            for u in range(_H_MLP // 16):
                tt = jnp.exp(jnp.minimum(s2 * w1g[u] + b2g[u], 60.0))
                macc = macc + ((tt - 1.0) / (tt + 1.0)) * w2g[u]
            scale_rows(e, _EPS_K * _hsum(macc))
            return 0

        lax.fori_loop(0, _C, edge, 0)
        scatter_fk()
        return 0

    lax.fori_loop(0, _NCH, hk_chunk, 0)
    lax.fori_loop(0, _NCH, k_chunk, 0)

    plsc.subcore_barrier()
    pltpu.sync_copy(fh_acc, fh_out.at[wid])
    pltpu.sync_copy(fk_sh.at[pl.ds(sid * _RPT, _RPT)],
                    fk_out.at[cid, pl.ds(sid * _RPT, _RPT)])


def _edge_phase(sK, g, weights, ind_K, ind_HK, w1, b1, w2):
    gp = jnp.pad(g, (0, _NP - _N))
    wflat = weights.reshape(-1)
    i0 = jnp.pad(ind_HK[:, 0], (0, 16))
    i1 = jnp.pad(ind_HK[:, 1], (0, 16))
    j0 = jnp.pad(ind_K[:, 0], (0, 16))
    j1 = jnp.pad(ind_K[:, 1], (0, 16))
    w1d = 2.0 * w1.reshape(_H_MLP)
    b2 = 2.0 * b1
    w2r = w2.reshape(_H_MLP)
    zfh = jnp.zeros((_NP,), jnp.float32)
    zfk = jnp.zeros((_RPT, _D), jnp.float32)

    run = pl.kernel(
        _edge_body,
        mesh=plsc.VectorSubcoreMesh(core_axis_name="c", subcore_axis_name="s"),
        out_type=[
            jax.ShapeDtypeStruct((32, _NP), jnp.float32),
            jax.ShapeDtypeStruct((2, _NP, _D), jnp.float32),
        ],
        scratch_types=[
            pltpu.VMEM((_NP,), jnp.float32),
            pltpu.VMEM((_NP,), jnp.float32),
            pltpu.VMEM((_C,), jnp.int32),
            pltpu.VMEM((_C,), jnp.int32),
            pltpu.VMEM((_C + 16,), jnp.int32),
            pltpu.VMEM((_C + 16,), jnp.int32),
            pltpu.VMEM((_C + 16,), jnp.int32),
            pltpu.VMEM((_C + 16,), jnp.int32),
            pltpu.VMEM((_C + 16,), jnp.float32),
            pltpu.VMEM((_C + 16,), jnp.float32),
            pltpu.VMEM((_C, _D), jnp.float32),
            pltpu.VMEM((_C, _D), jnp.float32),
            pltpu.VMEM((32,), jnp.float32),
            pltpu.VMEM((_H_MLP,), jnp.float32),
            pltpu.VMEM((_H_MLP,), jnp.float32),
            pltpu.VMEM((_H_MLP,), jnp.float32),
            pltpu.VMEM_SHARED((_NP, _D), jnp.float32),
            pltpu.SemaphoreType.DMA,
        ],
    )
    fh_part, fk_part = run(sK, gp, wflat, i0, i1, j0, j1,
                           w1d, b2, w2r, zfh, zfk)
    return fh_part.sum(axis=0)[:_N], fk_part.sum(axis=0)[:_N]


def kernel(t, state_H, state_K, ind_K, ind_HK, weights, omega, w1, b1, w2):
    sK = state_K / jnp.linalg.norm(state_K, axis=1, keepdims=True)
    g = jnp.tanh(state_H)
    Wg = _sym_matvec(weights, g)

    fh_edges, f_K = _edge_phase(sK, g, weights, ind_K, ind_HK, w1, b1, w2)
    f_H = -state_H + Wg + fh_edges

    f_K = (
        -f_K
        + sK * jnp.sum(sK * f_K, axis=1, keepdims=True)
        + sK @ ((omega - omega.T) / 2.0)
    )
    return (f_H, f_K)


# speedup vs baseline: 2.7646x; 1.0263x over previous
import functools

import jax
import jax.numpy as jnp
from jax import lax
from jax.experimental import pallas as pl
from jax.experimental.pallas import tpu as pltpu
from jax.experimental.pallas import tpu_sc as plsc

_N = 10000
_D = 128
_H_MLP = 64
_EPS_K = 0.1
_KAPPA_K = 1.0
_KAPPA_H = 1.0

_BM = 200


def _symmv_body(w_ref, grow_ref, gcol_ref, y1_ref, y2_ref):
    i = pl.program_id(0)

    @pl.when(i == 0)
    def _():
        y2_ref[...] = jnp.zeros_like(y2_ref)

    wblk = w_ref[...]
    y1 = jnp.dot(wblk, gcol_ref[...], preferred_element_type=jnp.float32)
    y1_ref[...] = y1.reshape(1, 1, _BM)
    grow = grow_ref[0, 0, :]
    y2_ref[...] += jnp.dot(grow, wblk, preferred_element_type=jnp.float32)[None, :]


def _sym_matvec(weights, g):
    ni = _N // _BM
    g3 = g.reshape(ni, 1, _BM)
    y1, y2 = pl.pallas_call(
        _symmv_body,
        grid=(ni,),
        in_specs=[
            pl.BlockSpec((_BM, _N), lambda i: (i, 0)),
            pl.BlockSpec((1, 1, _BM), lambda i: (i, 0, 0)),
            pl.BlockSpec((_N,), lambda i: (0,)),
        ],
        out_specs=[
            pl.BlockSpec((1, 1, _BM), lambda i: (i, 0, 0)),
            pl.BlockSpec((1, _N), lambda i: (0, 0)),
        ],
        out_shape=[
            jax.ShapeDtypeStruct((ni, 1, _BM), jnp.float32),
            jax.ShapeDtypeStruct((1, _N), jnp.float32),
        ],
    )(weights, g3, g)
    return 0.5 * (y1.reshape(_N) + y2.reshape(_N))


_NP = 10240
_EPT = 10000
_C = 80
_CP = 128
_NCH = _EPT // _C
_RPT = _NP // 16


_DUMP = 10100


def _edge_body(sk_hbm, g_hbm, wflat_hbm, i0_hbm, i1_hbm, j0_hbm, j1_hbm,
               w1d_hbm, b2_hbm, w2_hbm, zfh_hbm, zfk_hbm,
               fh_out, fk_out,
               g_tab, fh_acc, idx0, idx1, idxp0, idxp1,
               fl01, fl10, w01, w10,
               rows0, rows1, red_v, w1v, b2v, w2v, fk_sh, sem):
    cid = lax.axis_index("c")
    sid = lax.axis_index("s")
    wid = sid * 2 + cid

    pltpu.sync_copy(g_hbm, g_tab)
    pltpu.sync_copy(w1d_hbm, w1v)
    pltpu.sync_copy(b2_hbm, b2v)
    pltpu.sync_copy(w2_hbm, w2v)
    pltpu.sync_copy(zfh_hbm, fh_acc)
    pltpu.sync_copy(zfk_hbm, fk_sh.at[pl.ds(sid * _RPT, _RPT)])
    plsc.subcore_barrier()

    def _hsum(v, off=0):
        for sh in (8, 4, 2, 1):
            red_v[pl.ds(off, 16)] = v
            v = v + red_v[pl.ds(off + sh, 16)]
        return v[0]

    lane0 = lax.iota(jnp.int32, 16) == 0
    w1g = [w1v[pl.ds(16 * i, 16)] for i in range(_H_MLP // 16)]
    b2g = [b2v[pl.ds(16 * i, 16)] for i in range(_H_MLP // 16)]
    w2g = [w2v[pl.ds(16 * i, 16)] for i in range(_H_MLP // 16)]

    def gather_chunk(k, ia_hbm, ib_hbm, with_w):
        base = wid * _EPT + k * _C
        icps = [pltpu.async_copy(ia_hbm.at[pl.ds(base, _C)], idx0, sem),
                pltpu.async_copy(ib_hbm.at[pl.ds(base, _C)], idx1, sem),
                pltpu.async_copy(ia_hbm.at[pl.ds(base, _C + 16)], idxp0, sem),
                pltpu.async_copy(ib_hbm.at[pl.ds(base, _C + 16)], idxp1, sem)]
        for cp in icps:
            cp.wait()
        cps = [pltpu.async_copy(sk_hbm.at[idx0], rows0, sem),
               pltpu.async_copy(sk_hbm.at[idx1], rows1, sem)]
        if with_w:
            def mkflat(gi, _):
                a = idxp0[pl.ds(gi * 16, 16)]
                b = idxp1[pl.ds(gi * 16, 16)]
                fl01[pl.ds(gi * 16, 16)] = a * _N + b
                fl10[pl.ds(gi * 16, 16)] = b * _N + a
                return 0
            lax.fori_loop(0, (_C + 16) // 16, mkflat, 0)
            cps.append(pltpu.async_copy(wflat_hbm.at[fl01], w01, sem))
            cps.append(pltpu.async_copy(wflat_hbm.at[fl10], w10, sem))
        for cp in cps:
            cp.wait()

    def edge_dot(e, off=0):
        acc = rows0[e, pl.ds(0, 16)] * rows1[e, pl.ds(0, 16)]
        for kk in range(1, _D // 16):
            sl = pl.ds(kk * 16, 16)
            acc = acc + rows0[e, sl] * rows1[e, sl]
        return _hsum(acc, off)

    def scale_rows(e, ck):
        ckv = jnp.full((16,), ck, jnp.float32)
        for kk in range(_D // 16):
            sl = pl.ds(kk * 16, 16)
            rows1[e, sl] = rows1[e, sl] * ckv
            rows0[e, sl] = rows0[e, sl] * ckv

    def scatter_fk():
        pltpu.sync_copy(rows1, fk_sh.at[idx0], add=True)
        pltpu.sync_copy(rows0, fk_sh.at[idx1], add=True)

    def hk_edge(e, off=0):
        gram = edge_dot(e, off)
        i0s = idxp0[pl.ds(e, 16)][0]
        i1s = idxp1[pl.ds(e, 16)][0]
        g0 = g_tab[pl.ds(i0s, 16)][0]
        g1 = g_tab[pl.ds(i1s, 16)][0]
        we = 0.5 * (w01[pl.ds(e, 16)][0] + w10[pl.ds(e, 16)][0])
        gw = gram * we * (1.0 / _KAPPA_H)
        v0 = fh_acc[pl.ds(i0s, 16)]
        fh_acc[pl.ds(i0s, 16)] = v0 + jnp.where(lane0, gw * g1, 0.0)
        v1 = fh_acc[pl.ds(i1s, 16)]
        fh_acc[pl.ds(i1s, 16)] = v1 + jnp.where(lane0, gw * g0, 0.0)
        scale_rows(e, -(g0 * g1) * we * (1.0 / _KAPPA_K))

    def hk_chunk(k, _):
        gather_chunk(k, i0_hbm, i1_hbm, True)

        def edge2(e2, _):
            hk_edge(e2 * 2, 0)
            hk_edge(e2 * 2 + 1, 32)
            return 0

        lax.fori_loop(0, _C // 2, edge2, 0)
        scatter_fk()
        return 0

    def k_edge(e, off=0):
        s2 = 2.0 * edge_dot(e, off)
        macc = jnp.zeros((16,), jnp.float32)
        for u in range(_H_MLP // 16):
            tt = jnp.exp(jnp.minimum(s2 * w1g[u] + b2g[u], 60.0))
            macc = macc + ((tt - 1.0) / (tt + 1.0)) * w2g[u]
        scale_rows(e, _EPS_K * _hsum(macc, off))

    def k_chunk(k, _):
        gather_chunk(k, j0_hbm, j1_hbm, False)

        def edge2(e2, _):
            k_edge(e2 * 2, 0)
            k_edge(e2 * 2 + 1, 32)
            return 0

        lax.fori_loop(0, _C // 2, edge2, 0)
        scatter_fk()
        return 0

    lax.fori_loop(0, _NCH, hk_chunk, 0)
    lax.fori_loop(0, _NCH, k_chunk, 0)

    plsc.subcore_barrier()
    pltpu.sync_copy(fh_acc, fh_out.at[wid])
    pltpu.sync_copy(fk_sh.at[pl.ds(sid * _RPT, _RPT)],
                    fk_out.at[cid, pl.ds(sid * _RPT, _RPT)])


def _edge_phase(sK, g, weights, ind_K, ind_HK, w1, b1, w2):
    gp = jnp.pad(g, (0, _NP - _N))
    wflat = weights.reshape(-1)
    i0 = jnp.pad(ind_HK[:, 0], (0, 16))
    i1 = jnp.pad(ind_HK[:, 1], (0, 16))
    j0 = jnp.pad(ind_K[:, 0], (0, 16))
    j1 = jnp.pad(ind_K[:, 1], (0, 16))
    w1d = 2.0 * w1.reshape(_H_MLP)
    b2 = 2.0 * b1
    w2r = w2.reshape(_H_MLP)
    zfh = jnp.zeros((_NP,), jnp.float32)
    zfk = jnp.zeros((_RPT, _D), jnp.float32)

    run = pl.kernel(
        _edge_body,
        mesh=plsc.VectorSubcoreMesh(core_axis_name="c", subcore_axis_name="s"),
        out_type=[
            jax.ShapeDtypeStruct((32, _NP), jnp.float32),
            jax.ShapeDtypeStruct((2, _NP, _D), jnp.float32),
        ],
        scratch_types=[
            pltpu.VMEM((_NP,), jnp.float32),
            pltpu.VMEM((_NP,), jnp.float32),
            pltpu.VMEM((_C,), jnp.int32),
            pltpu.VMEM((_C,), jnp.int32),
            pltpu.VMEM((_C + 16,), jnp.int32),
            pltpu.VMEM((_C + 16,), jnp.int32),
            pltpu.VMEM((_C + 16,), jnp.int32),
            pltpu.VMEM((_C + 16,), jnp.int32),
            pltpu.VMEM((_C + 16,), jnp.float32),
            pltpu.VMEM((_C + 16,), jnp.float32),
            pltpu.VMEM((_C, _D), jnp.float32),
            pltpu.VMEM((_C, _D), jnp.float32),
            pltpu.VMEM((64,), jnp.float32),
            pltpu.VMEM((_H_MLP,), jnp.float32),
            pltpu.VMEM((_H_MLP,), jnp.float32),
            pltpu.VMEM((_H_MLP,), jnp.float32),
            pltpu.VMEM_SHARED((_NP, _D), jnp.float32),
            pltpu.SemaphoreType.DMA,
        ],
    )
    fh_part, fk_part = run(sK, gp, wflat, i0, i1, j0, j1,
                           w1d, b2, w2r, zfh, zfk)
    return fh_part.sum(axis=0)[:_N], fk_part.sum(axis=0)[:_N]


def kernel(t, state_H, state_K, ind_K, ind_HK, weights, omega, w1, b1, w2):
    sK = state_K / jnp.linalg.norm(state_K, axis=1, keepdims=True)
    g = jnp.tanh(state_H)
    Wg = _sym_matvec(weights, g)

    fh_edges, f_K = _edge_phase(sK, g, weights, ind_K, ind_HK, w1, b1, w2)
    f_H = -state_H + Wg + fh_edges

    f_K = (
        -f_K
        + sK * jnp.sum(sK * f_K, axis=1, keepdims=True)
        + sK @ ((omega - omega.T) / 2.0)
    )
    return (f_H, f_K)
